# Initial kernel scaffold; baseline (speedup 1.0000x reference)
import jax
import jax.numpy as jnp
from jax import lax
from jax.experimental import pallas as pl
from jax.experimental.pallas import tpu as pltpu
from jax.experimental.pallas import tpu_sc as plsc

N = 10000
E = 160000
D = 256

NC = 2
NS = 16
L = 16
HALF = N // NC
CHUNK = E // NS
B = 128
NB = (CHUNK + B - 1) // B
PADC = NB * B
DUMP = HALF
ACC_ROWS = 5120
DEN_ROWS = 320


def _tc_body(h_ref, w_ref, wl_ref, wr_ref, z_ref, el_ref, er_ref):
    zb = lax.dot_general(h_ref[...], w_ref[...],
                         (((1,), (1,)), ((), ())),
                         preferred_element_type=jnp.float32)
    z_ref[...] = zb
    el_ref[...] = jnp.sum(zb * wl_ref[...], axis=1, keepdims=True)
    er_ref[...] = jnp.sum(zb * wr_ref[...], axis=1, keepdims=True)


def _tc_part(h, W_fc, w_l, w_r):
    bn = 1000
    grid = (N // bn,)
    z, el, er = pl.pallas_call(
        _tc_body,
        grid=grid,
        in_specs=[
            pl.BlockSpec((bn, D), lambda i: (i, 0)),
            pl.BlockSpec((D, D), lambda i: (0, 0)),
            pl.BlockSpec((1, D), lambda i: (0, 0)),
            pl.BlockSpec((1, D), lambda i: (0, 0)),
        ],
        out_specs=[
            pl.BlockSpec((bn, D), lambda i: (i, 0)),
            pl.BlockSpec((bn, 1), lambda i: (i, 0)),
            pl.BlockSpec((bn, 1), lambda i: (i, 0)),
        ],
        out_shape=[
            jax.ShapeDtypeStruct((N, D), jnp.float32),
            jax.ShapeDtypeStruct((N, 1), jnp.float32),
            jax.ShapeDtypeStruct((N, 1), jnp.float32),
        ],
    )(h, W_fc, w_l, w_r)
    return z, el[:, 0], er[:, 0]


def _sc_body(src_hbm, dst_hbm, el_hbm, er_hbm, z_hbm, out_hbm,
             el_v, er_v, srcc, dstc, exv, dliv, alph,
             den2, rows, idxg, idxb, iota_ref,
             out_acc, den_acc, gsem):
    c = lax.axis_index("c")
    s = lax.axis_index("s")
    lo = c * HALF
    lane = lax.iota(jnp.int32, L)

    pltpu.sync_copy(el_hbm, el_v)
    pltpu.sync_copy(er_hbm.at[pl.ds(lo, HALF)], er_v)
    pltpu.sync_copy(src_hbm.at[pl.ds(s * CHUNK, CHUNK)],
                    srcc.at[pl.ds(0, CHUNK)])
    pltpu.sync_copy(dst_hbm.at[pl.ds(s * CHUNK, CHUNK)], dstc)

    def _zrow(r, _):
        for cc in range(D // L):
            rows[r, pl.ds(cc * L, L)] = jnp.zeros((L,), jnp.float32)
        return 0
    lax.fori_loop(0, B, _zrow, 0)

    def _zden(i, _):
        den2[i, :] = jnp.zeros((L,), jnp.float32)
        return 0
    lax.fori_loop(0, DEN_ROWS, _zden, 0)

    def _ziota(j, _):
        iota_ref[pl.ds(j * L, L)] = lane + j * L
        return 0
    lax.fori_loop(0, DEN_ROWS // L, _ziota, 0)

    for t in range((PADC - CHUNK) // L):
        tl = pl.ds(CHUNK + t * L, L)
        srcc[tl] = jnp.zeros((L,), jnp.int32)
        exv[tl] = jnp.zeros((L,), jnp.float32)
        dliv[tl] = jnp.full((L,), DUMP, jnp.int32)

    base = s * (ACC_ROWS // NS)
    pltpu.sync_copy(rows, out_acc.at[pl.ds(base, B)])
    pltpu.sync_copy(rows, out_acc.at[pl.ds(base + B, B)])
    pltpu.sync_copy(rows.at[pl.ds(0, 64)],
                    out_acc.at[pl.ds(base + 2 * B, 64)])

    @pl.when(s == 0)
    def _():
        pltpu.sync_copy(den2, den_acc)

    plsc.subcore_barrier()

    masks = [lane == k for k in range(L)]

    def _phase_a(i, _):
        sl = pl.ds(i * L, L)
        sv = srcc[sl]
        dv = dstc[sl]
        m = (dv >= lo) & (dv < lo + HALF)
        dl = jnp.where(m, dv - lo, DUMP)
        elg = plsc.load_gather(el_v, [sv])
        erg = plsc.load_gather(er_v, [jnp.where(m, dv - lo, 0)])
        e = elg + erg
        e = jnp.maximum(e, e * 0.01)
        ex = jnp.where(m, jnp.exp(e), 0.0)
        exv[sl] = ex
        dliv[sl] = dl
        dr = lax.shift_right_logical(dl, 4)
        dc = lax.bitwise_and(dl, L - 1)
        for k in range(L):
            plsc.addupdate_scatter(den2, [dr, dc], ex, mask=masks[k])
        return 0
    lax.fori_loop(0, CHUNK // L, _phase_a, 0)

    pltpu.sync_copy(den2, den_acc.at[iota_ref], add=True)
    plsc.subcore_barrier()
    pltpu.sync_copy(den_acc, den2)

    def _phase_c(i, _):
        sl = pl.ds(i * L, L)
        dl = dliv[sl]
        ex = exv[sl]
        dr = lax.shift_right_logical(dl, 4)
        dc = lax.bitwise_and(dl, L - 1)
        den = plsc.load_gather(den2, [dr, dc])
        alph[sl] = ex / (den + 1e-9)
        return 0
    lax.fori_loop(0, PADC // L, _phase_c, 0)

    def _phase_d(b, _):
        off = b * B
        pltpu.sync_copy(srcc.at[pl.ds(off, B)], idxg)
        pltpu.sync_copy(dliv.at[pl.ds(off, B)], idxb)
        pltpu.async_copy(z_hbm.at[idxg], rows, gsem).wait()

        def _scale(r, _):
            bav = plsc.load_gather(alph, [jnp.full((L,), off, jnp.int32) + r])
            for cc in range(D // L):
                cs = pl.ds(cc * L, L)
                rows[r, cs] = rows[r, cs] * bav
            return 0
        lax.fori_loop(0, B, _scale, 0)
        pltpu.sync_copy(rows, out_acc.at[idxb], add=True)
        return 0
    lax.fori_loop(0, NB, _phase_d, 0)

    plsc.subcore_barrier()

    @pl.when(s < 8)
    def _():
        rows_per = HALF // 8
        pltpu.sync_copy(out_acc.at[pl.ds(s * rows_per, rows_per)],
                        out_hbm.at[pl.ds(lo + s * rows_per, rows_per)])


def _sc_part(src, dst, el, er, z):
    mesh = plsc.VectorSubcoreMesh(core_axis_name="c", subcore_axis_name="s")
    f = pl.kernel(
        _sc_body,
        out_type=jax.ShapeDtypeStruct((N, D), jnp.float32),
        mesh=mesh,
        scratch_types=[
            pltpu.VMEM((N,), jnp.float32),
            pltpu.VMEM((HALF,), jnp.float32),
            pltpu.VMEM((PADC,), jnp.int32),
            pltpu.VMEM((CHUNK,), jnp.int32),
            pltpu.VMEM((PADC,), jnp.float32),
            pltpu.VMEM((PADC,), jnp.int32),
            pltpu.VMEM((PADC,), jnp.float32),
            pltpu.VMEM((DEN_ROWS, L), jnp.float32),
            pltpu.VMEM((B, D), jnp.float32),
            pltpu.VMEM((B,), jnp.int32),
            pltpu.VMEM((B,), jnp.int32),
            pltpu.VMEM((DEN_ROWS,), jnp.int32),
            pltpu.VMEM_SHARED((ACC_ROWS, D), jnp.float32),
            pltpu.VMEM_SHARED((DEN_ROWS, L), jnp.float32),
            pltpu.SemaphoreType.DMA,
        ],
    )
    return f(src, dst, el, er, z)


@jax.jit
def kernel(h, edge_index, W_fc, w_l, w_r):
    z, el, er = _tc_part(h, W_fc, w_l, w_r)
    src = edge_index[0]
    dst = edge_index[1]
    return _sc_part(src, dst, el, er, z)


# speedup vs baseline: 3.6809x; 3.6809x over previous
import jax
import jax.numpy as jnp
from jax import lax
from jax.experimental import pallas as pl
from jax.experimental.pallas import tpu as pltpu
from jax.experimental.pallas import tpu_sc as plsc

N = 10000
E = 160000
D = 256
NP = 2
DH = D // NP

NC = 2
NS = 16
L = 16
HALF = N // NC
CHUNK = E // NS
KB = 16
NBK = (CHUNK + KB - 1) // KB
PADC = NBK * KB
DUMP = HALF
ACC_ROWS = 5008
SLAB = ACC_ROWS // NS
DEN_ROWS = 48
DEN_LANES = 128


def _tc_body(h_ref, w_ref, wl_ref, wr_ref, z1_ref, z2_ref, el_ref, er_ref):
    zb = lax.dot_general(h_ref[...], w_ref[...],
                         (((1,), (1,)), ((), ())),
                         preferred_element_type=jnp.float32)
    z1_ref[...] = zb[:, :DH]
    z2_ref[...] = zb[:, DH:]
    el_ref[...] = jnp.sum(zb * wl_ref[...], axis=1, keepdims=True)
    er_ref[...] = jnp.sum(zb * wr_ref[...], axis=1, keepdims=True)


def _tc_part(h, W_fc, w_l, w_r):
    bn = 1000
    grid = (N // bn,)
    z1, z2, el, er = pl.pallas_call(
        _tc_body,
        grid=grid,
        in_specs=[
            pl.BlockSpec((bn, D), lambda i: (i, 0)),
            pl.BlockSpec((D, D), lambda i: (0, 0)),
            pl.BlockSpec((1, D), lambda i: (0, 0)),
            pl.BlockSpec((1, D), lambda i: (0, 0)),
        ],
        out_specs=[
            pl.BlockSpec((bn, DH), lambda i: (i, 0)),
            pl.BlockSpec((bn, DH), lambda i: (i, 0)),
            pl.BlockSpec((bn, 1), lambda i: (i, 0)),
            pl.BlockSpec((bn, 1), lambda i: (i, 0)),
        ],
        out_shape=[
            jax.ShapeDtypeStruct((N, DH), jnp.float32),
            jax.ShapeDtypeStruct((N, DH), jnp.float32),
            jax.ShapeDtypeStruct((N, 1), jnp.float32),
            jax.ShapeDtypeStruct((N, 1), jnp.float32),
        ],
    )(h, W_fc, w_l, w_r)
    return (z1, z2), el[:, 0], er[:, 0]


def _sc_body(src_hbm, dst_hbm, el_hbm, er_hbm, z1_hbm, z2_hbm, out_hbm,
             el_v, er_v, srcc, dstcl, exv,
             den2, gbuf, gidx, sidx, iota_ref,
             out_acc, den_acc, gsem):
    c = lax.axis_index("c")
    s = lax.axis_index("s")
    lo = c * HALF
    lane = lax.iota(jnp.int32, L)
    base = s * SLAB

    pltpu.sync_copy(el_hbm, el_v)
    pltpu.sync_copy(er_hbm.at[pl.ds(lo, HALF)], er_v)
    pltpu.sync_copy(src_hbm.at[pl.ds(s * CHUNK, CHUNK)],
                    srcc.at[pl.ds(0, CHUNK)])
    pltpu.sync_copy(dst_hbm.at[pl.ds(s * CHUNK, CHUNK)],
                    dstcl.at[pl.ds(0, CHUNK)])

    def _zero_gbuf():
        def _zrow(r, _):
            for cc in range(DH // L):
                gbuf[r, pl.ds(cc * L, L)] = jnp.zeros((L,), jnp.float32)
            return 0
        lax.fori_loop(0, KB, _zrow, 0)

    def _zero_acc_slab():
        for q in range(SLAB // KB):
            pltpu.sync_copy(gbuf, out_acc.at[pl.ds(base + q * KB, KB)])
        rem = SLAB % KB
        if rem:
            pltpu.sync_copy(gbuf.at[pl.ds(0, rem)],
                            out_acc.at[pl.ds(base + SLAB - rem, rem)])

    _zero_gbuf()
    _zero_acc_slab()

    def _zden(i, _):
        for cc in range(DEN_LANES // L):
            den2[i, pl.ds(cc * L, L)] = jnp.zeros((L,), jnp.float32)
        return 0
    lax.fori_loop(0, DEN_ROWS, _zden, 0)

    def _ziota(j, _):
        iota_ref[pl.ds(j * L, L)] = lane + j * L
        return 0
    lax.fori_loop(0, DEN_ROWS // L, _ziota, 0)

    for t in range((PADC - CHUNK) // L):
        fl = CHUNK + t * L
        srcc[pl.ds(fl, L)] = jnp.zeros((L,), jnp.int32)
        exv[pl.ds(fl, L)] = jnp.zeros((L,), jnp.float32)
        dstcl[pl.ds(fl, L)] = jnp.full((L,), DUMP, jnp.int32)

    @pl.when(s == 0)
    def _():
        pltpu.sync_copy(den2, den_acc)

    plsc.subcore_barrier()

    masks = [lane == k for k in range(L)]

    def _phase_a(i, _):
        sl = pl.ds(i * L, L)
        sv = srcc[sl]
        dv = dstcl[sl]
        m = (dv >= lo) & (dv < lo + HALF)
        dl = jnp.where(m, dv - lo, DUMP)
        elg = plsc.load_gather(el_v, [sv])
        erg = plsc.load_gather(er_v, [jnp.where(m, dv - lo, 0)])
        e = elg + erg
        e = jnp.maximum(e, e * 0.01)
        ex = jnp.where(m, jnp.exp(e), 0.0)
        exv[sl] = ex
        dstcl[sl] = dl
        dr = lax.shift_right_logical(dl, 7)
        dc = lax.bitwise_and(dl, DEN_LANES - 1)
        for k in range(L):
            plsc.addupdate_scatter(den2, [dr, dc], ex, mask=masks[k])
        return 0
    lax.fori_loop(0, CHUNK // L, _phase_a, 0)

    pltpu.sync_copy(den2, den_acc.at[iota_ref], add=True)
    plsc.subcore_barrier()
    pltpu.sync_copy(den_acc, den2)

    def _phase_c(i, _):
        sl = pl.ds(i * L, L)
        dl = dstcl[sl]
        ex = exv[sl]
        dr = lax.shift_right_logical(dl, 7)
        dc = lax.bitwise_and(dl, DEN_LANES - 1)
        den = plsc.load_gather(den2, [dr, dc])
        exv[sl] = ex / (den + 1e-9)
        return 0
    lax.fori_loop(0, PADC // L, _phase_c, 0)

    for p, z_hbm in enumerate((z1_hbm, z2_hbm)):
        def _phase_d(b, _):
            off = b * KB
            for u in range(KB // L):
                usl = pl.ds(off + u * L, L)
                gidx[pl.ds(u * L, L)] = srcc[usl]
                sidx[pl.ds(u * L, L)] = dstcl[usl]
            pltpu.async_copy(z_hbm.at[gidx], gbuf, gsem).wait()

            def _scale(r, _):
                bav = plsc.load_gather(
                    exv, [jnp.broadcast_to(off + r, (L,))])
                for cc in range(DH // L):
                    cs = pl.ds(cc * L, L)
                    gbuf[r, cs] = gbuf[r, cs] * bav
                return 0
            lax.fori_loop(0, KB, _scale, 0)
            pltpu.sync_copy(gbuf, out_acc.at[sidx], add=True)
            return 0
        lax.fori_loop(0, NBK, _phase_d, 0)

        plsc.subcore_barrier()

        @pl.when(s < 8)
        def _():
            pltpu.sync_copy(
                out_acc.at[pl.ds(s * 624, 624)],
                out_hbm.at[pl.ds(lo + s * 624, 624), pl.ds(p * DH, DH)])

        @pl.when(s == 8)
        def _():
            pltpu.sync_copy(
                out_acc.at[pl.ds(4992, 8)],
                out_hbm.at[pl.ds(lo + 4992, 8), pl.ds(p * DH, DH)])

        if p != NP - 1:
            plsc.subcore_barrier()
            _zero_gbuf()
            _zero_acc_slab()
            plsc.subcore_barrier()


def _sc_part(src, dst, el, er, zs):
    mesh = plsc.VectorSubcoreMesh(core_axis_name="c", subcore_axis_name="s")
    f = pl.kernel(
        _sc_body,
        out_type=jax.ShapeDtypeStruct((N, D), jnp.float32),
        mesh=mesh,
        compiler_params=pltpu.CompilerParams(needs_layout_passes=False),
        scratch_types=[
            pltpu.VMEM((N,), jnp.float32),
            pltpu.VMEM((HALF,), jnp.float32),
            pltpu.VMEM((PADC,), jnp.int32),
            pltpu.VMEM((PADC,), jnp.int32),
            pltpu.VMEM((PADC,), jnp.float32),
            pltpu.VMEM((DEN_ROWS, DEN_LANES), jnp.float32),
            pltpu.VMEM((KB, DH), jnp.float32),
            pltpu.VMEM((KB,), jnp.int32),
            pltpu.VMEM((KB,), jnp.int32),
            pltpu.VMEM((DEN_ROWS,), jnp.int32),
            pltpu.VMEM_SHARED((ACC_ROWS, DH), jnp.float32),
            pltpu.VMEM_SHARED((DEN_ROWS, DEN_LANES), jnp.float32),
            pltpu.SemaphoreType.DMA,
        ],
    )
    return f(src, dst, el, er, *zs)


@jax.jit
def kernel(h, edge_index, W_fc, w_l, w_r):
    zs, el, er = _tc_part(h, W_fc, w_l, w_r)
    src = edge_index[0]
    dst = edge_index[1]
    return _sc_part(src, dst, el, er, zs)


# speedup vs baseline: 7.0738x; 1.9218x over previous
import jax
import jax.numpy as jnp
from jax import lax
from jax.experimental import pallas as pl
from jax.experimental.pallas import tpu as pltpu
from jax.experimental.pallas import tpu_sc as plsc

N = 10000
E = 160000
D = 256
NP = 2
DH = D // NP

NC = 2
NS = 16
L = 16
HALF = N // NC
CHUNK = E // NS
KB = 80
NBK = (CHUNK + KB - 1) // KB
PADC = NBK * KB
DUMP = HALF
ACC_ROWS = 5008
SLAB = ACC_ROWS // NS
DEN_ROWS = 48
DEN_LANES = 128


def _tc_body(h_ref, w_ref, wl_ref, wr_ref, z1_ref, z2_ref, el_ref, er_ref):
    zb = lax.dot_general(h_ref[...], w_ref[...],
                         (((1,), (1,)), ((), ())),
                         preferred_element_type=jnp.float32)
    z1_ref[...] = zb[:, :DH]
    z2_ref[...] = zb[:, DH:]
    el_ref[...] = jnp.sum(zb * wl_ref[...], axis=1, keepdims=True)
    er_ref[...] = jnp.sum(zb * wr_ref[...], axis=1, keepdims=True)


def _tc_part(h, W_fc, w_l, w_r):
    bn = 1000
    grid = (N // bn,)
    z1, z2, el, er = pl.pallas_call(
        _tc_body,
        grid=grid,
        in_specs=[
            pl.BlockSpec((bn, D), lambda i: (i, 0)),
            pl.BlockSpec((D, D), lambda i: (0, 0)),
            pl.BlockSpec((1, D), lambda i: (0, 0)),
            pl.BlockSpec((1, D), lambda i: (0, 0)),
        ],
        out_specs=[
            pl.BlockSpec((bn, DH), lambda i: (i, 0)),
            pl.BlockSpec((bn, DH), lambda i: (i, 0)),
            pl.BlockSpec((bn, 1), lambda i: (i, 0)),
            pl.BlockSpec((bn, 1), lambda i: (i, 0)),
        ],
        out_shape=[
            jax.ShapeDtypeStruct((N, DH), jnp.float32),
            jax.ShapeDtypeStruct((N, DH), jnp.float32),
            jax.ShapeDtypeStruct((N, 1), jnp.float32),
            jax.ShapeDtypeStruct((N, 1), jnp.float32),
        ],
    )(h, W_fc, w_l, w_r)
    return (z1, z2), el[:, 0], er[:, 0]


def _sc_body(src_hbm, dst_hbm, el_hbm, er_hbm, z1_hbm, z2_hbm, out_hbm,
             el_v, er_v, srcc, dstcl, exv,
             den2, gbuf, gidx, sidx, iota_ref,
             out_acc, den_acc, gsem):
    c = lax.axis_index("c")
    s = lax.axis_index("s")
    lo = c * HALF
    lane = lax.iota(jnp.int32, L)
    base = s * SLAB

    pltpu.sync_copy(el_hbm, el_v)
    pltpu.sync_copy(er_hbm.at[pl.ds(lo, HALF)], er_v)
    pltpu.sync_copy(src_hbm.at[pl.ds(s * CHUNK, CHUNK)],
                    srcc.at[pl.ds(0, CHUNK)])
    pltpu.sync_copy(dst_hbm.at[pl.ds(s * CHUNK, CHUNK)],
                    dstcl.at[pl.ds(0, CHUNK)])

    def _zero_gbuf():
        def _zrow(r, _):
            for cc in range(DH // L):
                gbuf[r, pl.ds(cc * L, L)] = jnp.zeros((L,), jnp.float32)
            return 0
        lax.fori_loop(0, KB, _zrow, 0)

    def _zero_acc_slab():
        for q in range(SLAB // KB):
            pltpu.sync_copy(gbuf, out_acc.at[pl.ds(base + q * KB, KB)])
        rem = SLAB % KB
        if rem:
            pltpu.sync_copy(gbuf.at[pl.ds(0, rem)],
                            out_acc.at[pl.ds(base + SLAB - rem, rem)])

    _zero_gbuf()
    _zero_acc_slab()

    def _zden(i, _):
        for cc in range(DEN_LANES // L):
            den2[i, pl.ds(cc * L, L)] = jnp.zeros((L,), jnp.float32)
        return 0
    lax.fori_loop(0, DEN_ROWS, _zden, 0)

    def _ziota(j, _):
        iota_ref[pl.ds(j * L, L)] = lane + j * L
        return 0
    lax.fori_loop(0, DEN_ROWS // L, _ziota, 0)

    for t in range((PADC - CHUNK) // L):
        fl = CHUNK + t * L
        srcc[pl.ds(fl, L)] = jnp.zeros((L,), jnp.int32)
        exv[pl.ds(fl, L)] = jnp.zeros((L,), jnp.float32)
        dstcl[pl.ds(fl, L)] = jnp.full((L,), DUMP, jnp.int32)

    @pl.when(s == 0)
    def _():
        pltpu.sync_copy(den2, den_acc)

    plsc.subcore_barrier()

    masks = [lane == k for k in range(L)]

    def _phase_a(i, _):
        sl = pl.ds(i * L, L)
        sv = srcc[sl]
        dv = dstcl[sl]
        m = (dv >= lo) & (dv < lo + HALF)
        dl = jnp.where(m, dv - lo, DUMP)
        elg = plsc.load_gather(el_v, [sv])
        erg = plsc.load_gather(er_v, [jnp.where(m, dv - lo, 0)])
        e = elg + erg
        e = jnp.maximum(e, e * 0.01)
        ex = jnp.where(m, jnp.exp(e), 0.0)
        exv[sl] = ex
        dstcl[sl] = dl
        dr = lax.shift_right_logical(dl, 7)
        dc = lax.bitwise_and(dl, DEN_LANES - 1)
        for k in range(L):
            plsc.addupdate_scatter(den2, [dr, dc], ex, mask=masks[k])
        return 0
    lax.fori_loop(0, CHUNK // L, _phase_a, 0)

    pltpu.sync_copy(den2, den_acc.at[iota_ref], add=True)
    plsc.subcore_barrier()
    pltpu.sync_copy(den_acc, den2)

    def _phase_c(i, _):
        sl = pl.ds(i * L, L)
        dl = dstcl[sl]
        ex = exv[sl]
        dr = lax.shift_right_logical(dl, 7)
        dc = lax.bitwise_and(dl, DEN_LANES - 1)
        den = plsc.load_gather(den2, [dr, dc])
        exv[sl] = ex / (den + 1e-9)
        return 0
    lax.fori_loop(0, PADC // L, _phase_c, 0)

    for p, z_hbm in enumerate((z1_hbm, z2_hbm)):
        def _phase_d(b, _):
            off = b * KB
            for u in range(KB // L):
                usl = pl.ds(off + u * L, L)
                gidx[pl.ds(u * L, L)] = srcc[usl]
                sidx[pl.ds(u * L, L)] = dstcl[usl]
            pltpu.async_copy(z_hbm.at[gidx], gbuf, gsem).wait()

            def _scale(r, _):
                bav = plsc.load_gather(
                    exv, [jnp.broadcast_to(off + r, (L,))])
                for cc in range(DH // L):
                    cs = pl.ds(cc * L, L)
                    gbuf[r, cs] = gbuf[r, cs] * bav
                return 0
            lax.fori_loop(0, KB, _scale, 0)
            pltpu.sync_copy(gbuf, out_acc.at[sidx], add=True)
            return 0
        lax.fori_loop(0, NBK, _phase_d, 0)

        plsc.subcore_barrier()

        @pl.when(s < 8)
        def _():
            pltpu.sync_copy(
                out_acc.at[pl.ds(s * 624, 624)],
                out_hbm.at[pl.ds(lo + s * 624, 624), pl.ds(p * DH, DH)])

        @pl.when(s == 8)
        def _():
            pltpu.sync_copy(
                out_acc.at[pl.ds(4992, 8)],
                out_hbm.at[pl.ds(lo + 4992, 8), pl.ds(p * DH, DH)])

        if p != NP - 1:
            plsc.subcore_barrier()
            _zero_gbuf()
            _zero_acc_slab()
            plsc.subcore_barrier()


def _sc_part(src, dst, el, er, zs):
    mesh = plsc.VectorSubcoreMesh(core_axis_name="c", subcore_axis_name="s")
    f = pl.kernel(
        _sc_body,
        out_type=jax.ShapeDtypeStruct((N, D), jnp.float32),
        mesh=mesh,
        compiler_params=pltpu.CompilerParams(needs_layout_passes=False),
        scratch_types=[
            pltpu.VMEM((N,), jnp.float32),
            pltpu.VMEM((HALF,), jnp.float32),
            pltpu.VMEM((PADC,), jnp.int32),
            pltpu.VMEM((PADC,), jnp.int32),
            pltpu.VMEM((PADC,), jnp.float32),
            pltpu.VMEM((DEN_ROWS, DEN_LANES), jnp.float32),
            pltpu.VMEM((KB, DH), jnp.float32),
            pltpu.VMEM((KB,), jnp.int32),
            pltpu.VMEM((KB,), jnp.int32),
            pltpu.VMEM((DEN_ROWS,), jnp.int32),
            pltpu.VMEM_SHARED((ACC_ROWS, DH), jnp.float32),
            pltpu.VMEM_SHARED((DEN_ROWS, DEN_LANES), jnp.float32),
            pltpu.SemaphoreType.DMA,
        ],
    )
    return f(src, dst, el, er, *zs)


@jax.jit
def kernel(h, edge_index, W_fc, w_l, w_r):
    zs, el, er = _tc_part(h, W_fc, w_l, w_r)
    src = edge_index[0]
    dst = edge_index[1]
    return _sc_part(src, dst, el, er, zs)


# speedup vs baseline: 11.4405x; 1.6173x over previous
import jax
import jax.numpy as jnp
from jax import lax
from jax.experimental import pallas as pl
from jax.experimental.pallas import tpu as pltpu
from jax.experimental.pallas import tpu_sc as plsc

N = 10000
E = 160000
D = 256
NP = 2
DH = D // NP

NC = 2
NS = 16
L = 16
HALF = N // NC
CHUNK = E // NS
KB = 80
NBK = (CHUNK + KB - 1) // KB
PADC = NBK * KB
DUMP = HALF
ACC_ROWS = 5008
SLAB = ACC_ROWS // NS
DEN_ROWS = 48
DEN_LANES = 128


def _tc_body(h_ref, w_ref, wl_ref, wr_ref, z1_ref, z2_ref, el_ref, er_ref):
    zb = lax.dot_general(h_ref[...], w_ref[...],
                         (((1,), (1,)), ((), ())),
                         preferred_element_type=jnp.float32)
    z1_ref[...] = zb[:, :DH]
    z2_ref[...] = zb[:, DH:]
    el_ref[...] = jnp.sum(zb * wl_ref[...], axis=1, keepdims=True)
    er_ref[...] = jnp.sum(zb * wr_ref[...], axis=1, keepdims=True)


def _tc_part(h, W_fc, w_l, w_r):
    bn = 1000
    grid = (N // bn,)
    z1, z2, el, er = pl.pallas_call(
        _tc_body,
        grid=grid,
        in_specs=[
            pl.BlockSpec((bn, D), lambda i: (i, 0)),
            pl.BlockSpec((D, D), lambda i: (0, 0)),
            pl.BlockSpec((1, D), lambda i: (0, 0)),
            pl.BlockSpec((1, D), lambda i: (0, 0)),
        ],
        out_specs=[
            pl.BlockSpec((bn, DH), lambda i: (i, 0)),
            pl.BlockSpec((bn, DH), lambda i: (i, 0)),
            pl.BlockSpec((bn, 1), lambda i: (i, 0)),
            pl.BlockSpec((bn, 1), lambda i: (i, 0)),
        ],
        out_shape=[
            jax.ShapeDtypeStruct((N, DH), jnp.float32),
            jax.ShapeDtypeStruct((N, DH), jnp.float32),
            jax.ShapeDtypeStruct((N, 1), jnp.float32),
            jax.ShapeDtypeStruct((N, 1), jnp.float32),
        ],
    )(h, W_fc, w_l, w_r)
    return (z1, z2), el[:, 0], er[:, 0]


def _sc_body(src_hbm, dst_hbm, el_hbm, er_hbm, z1_hbm, z2_hbm, out_hbm,
             el_v, er_v, srcc, dstcl, exv,
             den2, gbuf, gidx, sidx, gbuf2, gidx2, sidx2, iota_ref,
             out_acc, den_acc, gsem, gsem2):
    c = lax.axis_index("c")
    s = lax.axis_index("s")
    lo = c * HALF
    lane = lax.iota(jnp.int32, L)
    base = s * SLAB

    pltpu.sync_copy(el_hbm, el_v)
    pltpu.sync_copy(er_hbm.at[pl.ds(lo, HALF)], er_v)
    pltpu.sync_copy(src_hbm.at[pl.ds(s * CHUNK, CHUNK)],
                    srcc.at[pl.ds(0, CHUNK)])
    pltpu.sync_copy(dst_hbm.at[pl.ds(s * CHUNK, CHUNK)],
                    dstcl.at[pl.ds(0, CHUNK)])

    def _zero_gbuf():
        def _zrow(r, _):
            for cc in range(DH // L):
                gbuf[r, pl.ds(cc * L, L)] = jnp.zeros((L,), jnp.float32)
            return 0
        lax.fori_loop(0, KB, _zrow, 0)

    def _zero_acc_slab():
        for q in range(SLAB // KB):
            pltpu.sync_copy(gbuf, out_acc.at[pl.ds(base + q * KB, KB)])
        rem = SLAB % KB
        if rem:
            pltpu.sync_copy(gbuf.at[pl.ds(0, rem)],
                            out_acc.at[pl.ds(base + SLAB - rem, rem)])

    _zero_gbuf()
    _zero_acc_slab()

    def _zden(i, _):
        for cc in range(DEN_LANES // L):
            den2[i, pl.ds(cc * L, L)] = jnp.zeros((L,), jnp.float32)
        return 0
    lax.fori_loop(0, DEN_ROWS, _zden, 0)

    def _ziota(j, _):
        iota_ref[pl.ds(j * L, L)] = lane + j * L
        return 0
    lax.fori_loop(0, DEN_ROWS // L, _ziota, 0)

    for t in range((PADC - CHUNK) // L):
        fl = CHUNK + t * L
        srcc[pl.ds(fl, L)] = jnp.zeros((L,), jnp.int32)
        exv[pl.ds(fl, L)] = jnp.zeros((L,), jnp.float32)
        dstcl[pl.ds(fl, L)] = jnp.full((L,), DUMP, jnp.int32)

    @pl.when(s == 0)
    def _():
        pltpu.sync_copy(den2, den_acc)

    plsc.subcore_barrier()

    masks = [lane == k for k in range(L)]

    def _phase_a(i, _):
        sl = pl.ds(i * L, L)
        sv = srcc[sl]
        dv = dstcl[sl]
        m = (dv >= lo) & (dv < lo + HALF)
        dl = jnp.where(m, dv - lo, DUMP)
        elg = plsc.load_gather(el_v, [sv])
        erg = plsc.load_gather(er_v, [jnp.where(m, dv - lo, 0)])
        e = elg + erg
        e = jnp.maximum(e, e * 0.01)
        ex = jnp.where(m, jnp.exp(e), 0.0)
        exv[sl] = ex
        dstcl[sl] = dl
        dr = lax.shift_right_logical(dl, 7)
        dc = lax.bitwise_and(dl, DEN_LANES - 1)
        for k in range(L):
            plsc.addupdate_scatter(den2, [dr, dc], ex, mask=masks[k])
        return 0
    lax.fori_loop(0, CHUNK // L, _phase_a, 0)

    pltpu.sync_copy(den2, den_acc.at[iota_ref], add=True)
    plsc.subcore_barrier()
    pltpu.sync_copy(den_acc, den2)

    def _phase_c(i, _):
        sl = pl.ds(i * L, L)
        dl = dstcl[sl]
        ex = exv[sl]
        dr = lax.shift_right_logical(dl, 7)
        dc = lax.bitwise_and(dl, DEN_LANES - 1)
        den = plsc.load_gather(den2, [dr, dc])
        exv[sl] = ex / (den + 1e-9)
        return 0
    lax.fori_loop(0, PADC // L, _phase_c, 0)

    for p, z_hbm in enumerate((z1_hbm, z2_hbm)):
        def _fill(gi, si, off):
            for u in range(KB // L):
                usl = pl.ds(off + u * L, L)
                gi[pl.ds(u * L, L)] = srcc[usl]
                si[pl.ds(u * L, L)] = dstcl[usl]

        def _start(gi, gb, sem):
            pltpu.make_async_copy(z_hbm.at[gi], gb, sem).start()

        def _wait(gi, gb, sem):
            pltpu.make_async_copy(z_hbm.at[gi], gb, sem).wait()

        def _proc(gb, si, off):
            def _scale(r, _):
                bav = plsc.load_gather(
                    exv, [jnp.broadcast_to(off + r, (L,))])
                for cc in range(DH // L):
                    cs = pl.ds(cc * L, L)
                    gb[r, cs] = gb[r, cs] * bav
                return 0
            lax.fori_loop(0, KB, _scale, 0)
            pltpu.sync_copy(gb, out_acc.at[si], add=True)

        _fill(gidx, sidx, 0)
        _start(gidx, gbuf, gsem)

        def _pair(i, _):
            offa = (2 * i) * KB
            offb = offa + KB
            _fill(gidx2, sidx2, offb)
            _start(gidx2, gbuf2, gsem2)
            _wait(gidx, gbuf, gsem)
            _proc(gbuf, sidx, offa)
            _fill(gidx, sidx, offa + 2 * KB)
            _start(gidx, gbuf, gsem)
            _wait(gidx2, gbuf2, gsem2)
            _proc(gbuf2, sidx2, offb)
            return 0
        lax.fori_loop(0, NBK // 2, _pair, 0)

        _wait(gidx, gbuf, gsem)
        _proc(gbuf, sidx, (NBK - 1) * KB)

        plsc.subcore_barrier()

        @pl.when(s < 8)
        def _():
            pltpu.sync_copy(
                out_acc.at[pl.ds(s * 624, 624)],
                out_hbm.at[pl.ds(lo + s * 624, 624), pl.ds(p * DH, DH)])

        @pl.when(s == 8)
        def _():
            pltpu.sync_copy(
                out_acc.at[pl.ds(4992, 8)],
                out_hbm.at[pl.ds(lo + 4992, 8), pl.ds(p * DH, DH)])

        if p != NP - 1:
            plsc.subcore_barrier()
            _zero_gbuf()
            _zero_acc_slab()
            plsc.subcore_barrier()


def _sc_part(src, dst, el, er, zs):
    mesh = plsc.VectorSubcoreMesh(core_axis_name="c", subcore_axis_name="s")
    f = pl.kernel(
        _sc_body,
        out_type=jax.ShapeDtypeStruct((N, D), jnp.float32),
        mesh=mesh,
        compiler_params=pltpu.CompilerParams(needs_layout_passes=False),
        scratch_types=[
            pltpu.VMEM((N,), jnp.float32),
            pltpu.VMEM((HALF,), jnp.float32),
            pltpu.VMEM((PADC,), jnp.int32),
            pltpu.VMEM((PADC,), jnp.int32),
            pltpu.VMEM((PADC,), jnp.float32),
            pltpu.VMEM((DEN_ROWS, DEN_LANES), jnp.float32),
            pltpu.VMEM((KB, DH), jnp.float32),
            pltpu.VMEM((KB,), jnp.int32),
            pltpu.VMEM((KB,), jnp.int32),
            pltpu.VMEM((KB, DH), jnp.float32),
            pltpu.VMEM((KB,), jnp.int32),
            pltpu.VMEM((KB,), jnp.int32),
            pltpu.VMEM((DEN_ROWS,), jnp.int32),
            pltpu.VMEM_SHARED((ACC_ROWS, DH), jnp.float32),
            pltpu.VMEM_SHARED((DEN_ROWS, DEN_LANES), jnp.float32),
            pltpu.SemaphoreType.DMA,
            pltpu.SemaphoreType.DMA,
        ],
    )
    return f(src, dst, el, er, *zs)


@jax.jit
def kernel(h, edge_index, W_fc, w_l, w_r):
    zs, el, er = _tc_part(h, W_fc, w_l, w_r)
    src = edge_index[0]
    dst = edge_index[1]
    return _sc_part(src, dst, el, er, zs)


# speedup vs baseline: 18.3398x; 1.6031x over previous
import jax
import jax.numpy as jnp
from jax import lax
from jax.experimental import pallas as pl
from jax.experimental.pallas import tpu as pltpu
from jax.experimental.pallas import tpu_sc as plsc

N = 10000
E = 160000
D = 256
NP = 2
DH = D // NP

NC = 2
NS = 16
L = 16
HALF = N // NC
CHUNK = E // NS
KB = 80
NBK = (CHUNK + KB - 1) // KB
PADC = NBK * KB
DUMP = HALF
ACC_ROWS = 5008
SLAB = ACC_ROWS // NS
DEN_ROWS = 48
DEN_LANES = 128


def _tc_body(h_ref, w_ref, wl_ref, wr_ref, z1_ref, z2_ref, el_ref, er_ref):
    zb = lax.dot_general(h_ref[...], w_ref[...],
                         (((1,), (1,)), ((), ())),
                         preferred_element_type=jnp.float32)
    z1_ref[...] = zb[:, :DH]
    z2_ref[...] = zb[:, DH:]
    el_ref[...] = jnp.sum(zb * wl_ref[...], axis=1, keepdims=True)
    er_ref[...] = jnp.sum(zb * wr_ref[...], axis=1, keepdims=True)


def _tc_part(h, W_fc, w_l, w_r):
    bn = 1000
    grid = (N // bn,)
    z1, z2, el, er = pl.pallas_call(
        _tc_body,
        grid=grid,
        in_specs=[
            pl.BlockSpec((bn, D), lambda i: (i, 0)),
            pl.BlockSpec((D, D), lambda i: (0, 0)),
            pl.BlockSpec((1, D), lambda i: (0, 0)),
            pl.BlockSpec((1, D), lambda i: (0, 0)),
        ],
        out_specs=[
            pl.BlockSpec((bn, DH), lambda i: (i, 0)),
            pl.BlockSpec((bn, DH), lambda i: (i, 0)),
            pl.BlockSpec((bn, 1), lambda i: (i, 0)),
            pl.BlockSpec((bn, 1), lambda i: (i, 0)),
        ],
        out_shape=[
            jax.ShapeDtypeStruct((N, DH), jnp.float32),
            jax.ShapeDtypeStruct((N, DH), jnp.float32),
            jax.ShapeDtypeStruct((N, 1), jnp.float32),
            jax.ShapeDtypeStruct((N, 1), jnp.float32),
        ],
    )(h, W_fc, w_l, w_r)
    return (z1, z2), el[:, 0], er[:, 0]


def _sc_body(src_hbm, dst_hbm, el_hbm, er_hbm, z1_hbm, z2_hbm, out_hbm,
             el_v, er_v, srcc, dstcl, exv,
             den2, gbuf, gidx, sidx, gbuf2, gidx2, sidx2, iota_ref,
             out_acc, den_acc, gsem, gsem2):
    c = lax.axis_index("c")
    s = lax.axis_index("s")
    lo = c * HALF
    lane = lax.iota(jnp.int32, L)
    base = s * SLAB

    pltpu.sync_copy(el_hbm, el_v)
    pltpu.sync_copy(er_hbm.at[pl.ds(lo, HALF)], er_v)
    pltpu.sync_copy(src_hbm.at[pl.ds(s * CHUNK, CHUNK)],
                    srcc.at[pl.ds(0, CHUNK)])
    pltpu.sync_copy(dst_hbm.at[pl.ds(s * CHUNK, CHUNK)],
                    dstcl.at[pl.ds(0, CHUNK)])

    def _zero_gbuf():
        def _zrow(r, _):
            for cc in range(DH // L):
                gbuf[r, pl.ds(cc * L, L)] = jnp.zeros((L,), jnp.float32)
            return 0
        lax.fori_loop(0, KB, _zrow, 0)

    def _zero_acc_slab():
        for q in range(SLAB // KB):
            pltpu.sync_copy(gbuf, out_acc.at[pl.ds(base + q * KB, KB)])
        rem = SLAB % KB
        if rem:
            pltpu.sync_copy(gbuf.at[pl.ds(0, rem)],
                            out_acc.at[pl.ds(base + SLAB - rem, rem)])

    _zero_gbuf()
    _zero_acc_slab()

    def _zden(i, _):
        for cc in range(DEN_LANES // L):
            den2[i, pl.ds(cc * L, L)] = jnp.zeros((L,), jnp.float32)
        return 0
    lax.fori_loop(0, DEN_ROWS, _zden, 0)

    def _ziota(j, _):
        iota_ref[pl.ds(j * L, L)] = lane + j * L
        return 0
    lax.fori_loop(0, DEN_ROWS // L, _ziota, 0)

    for t in range((PADC - CHUNK) // L):
        fl = CHUNK + t * L
        srcc[pl.ds(fl, L)] = jnp.zeros((L,), jnp.int32)
        exv[pl.ds(fl, L)] = jnp.zeros((L,), jnp.float32)
        dstcl[pl.ds(fl, L)] = jnp.full((L,), DUMP, jnp.int32)

    @pl.when(s == 0)
    def _():
        pltpu.sync_copy(den2, den_acc)

    plsc.subcore_barrier()

    masks = [lane == k for k in range(L)]

    def _phase_a(i, cnt):
        sl = pl.ds(i * L, L)
        sv = srcc[sl]
        dv = dstcl[sl]
        m = (dv >= lo) & (dv < lo + HALF)
        dl = jnp.where(m, dv - lo, DUMP)
        elg = plsc.load_gather(el_v, [sv])
        erg = plsc.load_gather(er_v, [jnp.where(m, dv - lo, 0)])
        e = elg + erg
        e = jnp.maximum(e, e * 0.01)
        ex = jnp.where(m, jnp.exp(e), 0.0)
        dr = lax.shift_right_logical(dl, 7)
        dc = lax.bitwise_and(dl, DEN_LANES - 1)
        for k in range(L):
            plsc.addupdate_scatter(den2, [dr, dc], ex, mask=masks[k])
        plsc.store_compressed(srcc.at[pl.ds(cnt, L)], sv, mask=m)
        plsc.store_compressed(dstcl.at[pl.ds(cnt, L)], dl, mask=m)
        plsc.store_compressed(exv.at[pl.ds(cnt, L)], ex, mask=m)
        return cnt + jnp.sum(m.astype(jnp.int32))
    cnt = lax.fori_loop(0, CHUNK // L, _phase_a, jnp.int32(0))

    zf = jnp.zeros((L,), jnp.float32)
    dmp = jnp.full((L,), DUMP, jnp.int32)
    for v in range(KB // L):
        exv[pl.ds(cnt + v * L, L)] = zf
        dstcl[pl.ds(cnt + v * L, L)] = dmp

    nblocks = (cnt + (KB - 1)) // KB

    pltpu.sync_copy(den2, den_acc.at[iota_ref], add=True)
    plsc.subcore_barrier()
    pltpu.sync_copy(den_acc, den2)

    def _phase_c(i, _):
        sl = pl.ds(i * L, L)
        dl = dstcl[sl]
        ex = exv[sl]
        dr = lax.shift_right_logical(dl, 7)
        dc = lax.bitwise_and(dl, DEN_LANES - 1)
        den = plsc.load_gather(den2, [dr, dc])
        exv[sl] = ex / (den + 1e-9)
        return 0
    lax.fori_loop(0, nblocks * (KB // L), _phase_c, 0)

    for p, z_hbm in enumerate((z1_hbm, z2_hbm)):
        def _fill(gi, si, off):
            for u in range(KB // L):
                usl = pl.ds(off + u * L, L)
                gi[pl.ds(u * L, L)] = srcc[usl]
                si[pl.ds(u * L, L)] = dstcl[usl]

        def _start(gi, gb, sem):
            pltpu.make_async_copy(z_hbm.at[gi], gb, sem).start()

        def _wait(gi, gb, sem):
            pltpu.make_async_copy(z_hbm.at[gi], gb, sem).wait()

        def _proc(gb, si, off):
            def _scale(r, _):
                bav = plsc.load_gather(
                    exv, [jnp.broadcast_to(off + r, (L,))])
                for cc in range(DH // L):
                    cs = pl.ds(cc * L, L)
                    gb[r, cs] = gb[r, cs] * bav
                return 0
            lax.fori_loop(0, KB, _scale, 0)
            pltpu.sync_copy(gb, out_acc.at[si], add=True)

        @pl.when(nblocks > 0)
        def _():
            _fill(gidx, sidx, 0)
            _start(gidx, gbuf, gsem)

        def _pair(i, _):
            offa = (2 * i) * KB
            offb = offa + KB
            _fill(gidx2, sidx2, offb)
            _start(gidx2, gbuf2, gsem2)
            _wait(gidx, gbuf, gsem)
            _proc(gbuf, sidx, offa)

            @pl.when(2 * i + 2 < nblocks)
            def _():
                _fill(gidx, sidx, offa + 2 * KB)
                _start(gidx, gbuf, gsem)

            _wait(gidx2, gbuf2, gsem2)
            _proc(gbuf2, sidx2, offb)
            return 0
        lax.fori_loop(0, nblocks // 2, _pair, 0)

        @pl.when(lax.rem(nblocks, 2) == 1)
        def _():
            _wait(gidx, gbuf, gsem)
            _proc(gbuf, sidx, (nblocks - 1) * KB)

        plsc.subcore_barrier()

        @pl.when(s < 8)
        def _():
            pltpu.sync_copy(
                out_acc.at[pl.ds(s * 624, 624)],
                out_hbm.at[pl.ds(lo + s * 624, 624), pl.ds(p * DH, DH)])

        @pl.when(s == 8)
        def _():
            pltpu.sync_copy(
                out_acc.at[pl.ds(4992, 8)],
                out_hbm.at[pl.ds(lo + 4992, 8), pl.ds(p * DH, DH)])

        if p != NP - 1:
            plsc.subcore_barrier()
            _zero_gbuf()
            _zero_acc_slab()
            plsc.subcore_barrier()


def _sc_part(src, dst, el, er, zs):
    mesh = plsc.VectorSubcoreMesh(core_axis_name="c", subcore_axis_name="s")
    f = pl.kernel(
        _sc_body,
        out_type=jax.ShapeDtypeStruct((N, D), jnp.float32),
        mesh=mesh,
        compiler_params=pltpu.CompilerParams(needs_layout_passes=False),
        scratch_types=[
            pltpu.VMEM((N,), jnp.float32),
            pltpu.VMEM((HALF,), jnp.float32),
            pltpu.VMEM((PADC + KB,), jnp.int32),
            pltpu.VMEM((PADC + KB,), jnp.int32),
            pltpu.VMEM((PADC + KB,), jnp.float32),
            pltpu.VMEM((DEN_ROWS, DEN_LANES), jnp.float32),
            pltpu.VMEM((KB, DH), jnp.float32),
            pltpu.VMEM((KB,), jnp.int32),
            pltpu.VMEM((KB,), jnp.int32),
            pltpu.VMEM((KB, DH), jnp.float32),
            pltpu.VMEM((KB,), jnp.int32),
            pltpu.VMEM((KB,), jnp.int32),
            pltpu.VMEM((DEN_ROWS,), jnp.int32),
            pltpu.VMEM_SHARED((ACC_ROWS, DH), jnp.float32),
            pltpu.VMEM_SHARED((DEN_ROWS, DEN_LANES), jnp.float32),
            pltpu.SemaphoreType.DMA,
            pltpu.SemaphoreType.DMA,
        ],
    )
    return f(src, dst, el, er, *zs)


@jax.jit
def kernel(h, edge_index, W_fc, w_l, w_r):
    zs, el, er = _tc_part(h, W_fc, w_l, w_r)
    src = edge_index[0]
    dst = edge_index[1]
    return _sc_part(src, dst, el, er, zs)
